# baseline (speedup 1.0000x reference)
import functools

import jax
import jax.numpy as jnp
from jax import lax
from jax.experimental import pallas as pl
from jax.experimental.pallas import tpu as pltpu
from jax.experimental.pallas import tpu_sc as plsc

_SUBS = 16
_CHUNK = 128


def _elu(z):
    return jnp.where(z > 0, z, jnp.exp(jnp.minimum(z, 0.0)) - 1.0)


def _sc_scatter_add(n_acc, d_row, cpt0, cpt1):
    rows_per_sub = n_acc // _SUBS
    mesh = plsc.VectorSubcoreMesh(core_axis_name="c", subcore_axis_name="s")

    gb = 8

    @functools.partial(
        pl.kernel,
        mesh=mesh,
        out_type=[jax.ShapeDtypeStruct((n_acc, d_row), jnp.float32),
                  jax.ShapeDtypeStruct((n_acc, d_row), jnp.float32)],
        scratch_types=[
            pltpu.VMEM_SHARED((n_acc, d_row), jnp.float32),
            pltpu.VMEM((gb, _CHUNK), jnp.int32),
            pltpu.VMEM((gb, _CHUNK), jnp.int32),
            pltpu.VMEM((_CHUNK, d_row), jnp.float32),
            pltpu.VMEM((_CHUNK, d_row), jnp.float32),
            pltpu.SemaphoreType.DMA,
            pltpu.SemaphoreType.DMA,
            pltpu.SemaphoreType.DMA,
        ],
    )
    def k(h0, h1, src2d, dst2d, out0, out1, acc, idx_s, idx_d,
          rows0, rows1, sem_g0, sem_g1, sem_s):
        rows_bufs = (rows0, rows1)
        sem_g = (sem_g0, sem_g1)
        c = lax.axis_index("c")
        s = lax.axis_index("s")
        base = jnp.where(c == 0, s * cpt0, _SUBS * cpt0 + s * cpt1)
        n_groups = jnp.where(c == 0, cpt0 // gb, cpt1 // gb)

        def zb(r, carry):
            for u in range(d_row // 16):
                rows0[r, pl.ds(u * 16, 16)] = jnp.zeros((16,), jnp.float32)
            return carry
        lax.fori_loop(0, _CHUNK, zb, 0)
        for t in range(rows_per_sub // _CHUNK):
            pltpu.sync_copy(rows0,
                            acc.at[pl.ds(s * rows_per_sub + t * _CHUNK,
                                         _CHUNK)])

        plsc.subcore_barrier()

        def body(grp, carry):
            gbase = base + grp * gb
            pltpu.sync_copy(src2d.at[pl.ds(gbase, gb)], idx_s)
            pltpu.sync_copy(dst2d.at[pl.ds(gbase, gb)], idx_d)
            for j in range(2):
                pltpu.async_copy(h0.at[idx_s.at[j]], rows_bufs[j],
                                 sem_g[j])
            for j in range(gb):
                b = j % 2
                pltpu.make_async_copy(h0.at[idx_s.at[j]], rows_bufs[b],
                                      sem_g[b]).wait()
                pltpu.sync_copy(rows_bufs[b], acc.at[idx_d.at[j]],
                                add=True)
                if j + 2 < gb:
                    pltpu.async_copy(h0.at[idx_s.at[j + 2]],
                                     rows_bufs[b], sem_g[b])
            return carry

        lax.fori_loop(0, n_groups, body, 0)

        plsc.subcore_barrier()

        @pl.when(c == 0)
        def _():
            pltpu.sync_copy(acc.at[pl.ds(s * rows_per_sub, rows_per_sub)],
                            out0.at[pl.ds(s * rows_per_sub, rows_per_sub)])

        @pl.when(c == 1)
        def _():
            pltpu.sync_copy(acc.at[pl.ds(s * rows_per_sub, rows_per_sub)],
                            out1.at[pl.ds(s * rows_per_sub, rows_per_sub)])

    return k


def _tc_mlp(n, din, hmid, hout, extra_elu, block,
            first_matmul=True, post_w_cols=None):
    def body(*refs):
        if first_matmul:
            x, a, b, w1, b1, w2, b2 = refs[:7]
            nin = 7
        else:
            x, a, b, b1, w2, b2 = refs[:6]
            nin = 6
        xx = x[...] + a[...] + b[...]
        if first_matmul:
            t = jnp.dot(xx, w1[...], preferred_element_type=jnp.float32)
        else:
            t = xx
        h = _elu(t + b1[...])
        h = _elu(jnp.dot(h, w2[...], preferred_element_type=jnp.float32)
                 + b2[...])
        if extra_elu:
            h = _elu(h)
        if post_w_cols is not None:
            wp = refs[nin]
            h = jnp.dot(h, wp[...], preferred_element_type=jnp.float32)
        refs[-1][...] = h

    def row_spec(dcols):
        return pl.BlockSpec((block, dcols), lambda i: (i, 0))

    def full(shape):
        return pl.BlockSpec(shape, lambda i: (0,) * len(shape))

    in_specs = [row_spec(din)] * 3
    if first_matmul:
        in_specs += [full((din, hmid)), full((1, hmid))]
    else:
        in_specs += [full((1, hmid))]
    in_specs += [full((hmid, hout)), full((1, hout))]
    ocols = hout
    if post_w_cols is not None:
        in_specs.append(full((hout, post_w_cols)))
        ocols = post_w_cols

    return pl.pallas_call(
        body,
        grid=(n // block,),
        in_specs=in_specs,
        out_specs=row_spec(ocols),
        out_shape=jax.ShapeDtypeStruct((n, ocols), jnp.float32),
    )


def _tc_pool_mlp(n, d, g, dmid, block):
    nb = n // block
    dh = d // 2

    def body(hr, seg_ref, wm1, bm1, wm2, bm2, wm3, bm3, o, sums, counts):
        i = pl.program_id(0)

        @pl.when(i == 0)
        def _():
            sums[...] = jnp.zeros_like(sums)
            counts[...] = jnp.zeros_like(counts)

        seg = seg_ref[0, 0, :]
        oh = (seg[:, None]
              == lax.broadcasted_iota(jnp.int32, (block, g), 1)).astype(jnp.float32)
        h = hr[...]
        dn = (((0,), (0,)), ((), ()))
        sums[...] += lax.dot_general(oh, h, dn,
                                     preferred_element_type=jnp.float32)
        counts[...] += lax.dot_general(oh, jnp.ones((block, d), jnp.float32),
                                       dn, preferred_element_type=jnp.float32)

        @pl.when(i == nb - 1)
        def _():
            pooled = sums[...] / jnp.maximum(counts[...], 1.0)
            t = _elu(jnp.dot(pooled, wm1[...],
                                   preferred_element_type=jnp.float32) + bm1[...])
            t = _elu(jnp.dot(t, wm2[...],
                                   preferred_element_type=jnp.float32) + bm2[...])
            t = _elu(jnp.dot(t, wm3[...],
                                   preferred_element_type=jnp.float32) + bm3[...])
            o[...] = t

    def row_spec(dcols):
        return pl.BlockSpec((block, dcols), lambda i: (i, 0))

    def full(shape):
        return pl.BlockSpec(shape, lambda i: (0,) * len(shape))

    return pl.pallas_call(
        body,
        grid=(nb,),
        in_specs=[row_spec(d),
                  pl.BlockSpec((1, 1, block), lambda i: (i, 0, 0)),
                  full((d, dmid)), full((1, dmid)), full((dmid, d)),
                  full((1, d)), full((d, d)), full((1, d))],
        out_specs=pl.BlockSpec((g, d), lambda i: (0, 0)),
        out_shape=jax.ShapeDtypeStruct((g, d), jnp.float32),
        scratch_shapes=[pltpu.VMEM((g, d), jnp.float32),
                        pltpu.VMEM((g, d), jnp.float32)],
    )


def kernel(x, edge_index, batch,
           w1_1, b1_1, w1_2, b1_2,
           w2_1, b2_1, w2_2, b2_2,
           w3_1, b3_1, w3_2, b3_2,
           wm1, bm1, wm2, bm2, wm3, bm3):
    n, d = x.shape
    e = edge_index.shape[1]
    g = 64
    block = 1000

    ec = 2 * _SUBS * _CHUNK * 8
    epad = ((e + ec - 1) // ec) * ec
    cpt_sum = epad // (_SUBS * _CHUNK)
    cpt0 = cpt_sum
    cpt1 = 0
    nc = _SUBS * _CHUNK
    n_acc = ((n + nc - 1) // nc) * nc
    if n_acc == n:
        n_acc += nc
    src_p = jnp.concatenate(
        [edge_index[0], jnp.zeros((epad - e,), jnp.int32)]).reshape(-1, _CHUNK)
    dst_p = jnp.concatenate(
        [edge_index[1], jnp.full((epad - e,), n, jnp.int32)]).reshape(-1, _CHUNK)

    sc_full = _sc_scatter_add(n_acc, d, cpt0, cpt1)

    a1, p1 = sc_full(x, x, src_p, dst_p)
    h1 = _tc_mlp(n, 128, 128, 128, True, block)(
        x, a1, p1, w1_1, b1_1.reshape(1, -1), w1_2, b1_2.reshape(1, -1))

    a2, p2 = sc_full(h1, h1, src_p, dst_p)
    y2 = _tc_mlp(n, 128, 256, 256, True, block, post_w_cols=128)(
        h1, a2, p2, w2_1, b2_1.reshape(1, -1), w2_2, b2_2.reshape(1, -1),
        w3_1)

    a3, p3 = sc_full(y2, y2, src_p, dst_p)
    h3 = _tc_mlp(n, 128, 128, 128, False, block, first_matmul=False)(
        y2, a3, p3, b3_1.reshape(1, -1), w3_2, b3_2.reshape(1, -1))

    o = _tc_pool_mlp(n, 128, g, 256, block)(
        h3, batch.reshape(n // block, 1, block),
        wm1, bm1.reshape(1, -1), wm2, bm2.reshape(1, -1),
        wm3, bm3.reshape(1, -1))
    return o


# speedup vs baseline: 3.9798x; 3.9798x over previous
import functools

import jax
import jax.numpy as jnp
from jax import lax
from jax.experimental import pallas as pl
from jax.experimental.pallas import tpu as pltpu
from jax.experimental.pallas import tpu_sc as plsc

_SUBS = 16
_CHUNK = 128


def _elu(z):
    return jnp.where(z > 0, z, jnp.exp(jnp.minimum(z, 0.0)) - 1.0)


def _sc_scatter_add(n_acc, d_row, cpt0, cpt1):
    rows_per_sub = n_acc // _SUBS
    mesh = plsc.VectorSubcoreMesh(core_axis_name="c", subcore_axis_name="s")

    gb = 8

    @functools.partial(
        pl.kernel,
        mesh=mesh,
        out_type=[jax.ShapeDtypeStruct((n_acc, d_row), jnp.float32),
                  jax.ShapeDtypeStruct((n_acc, d_row), jnp.float32)],
        scratch_types=[
            pltpu.VMEM_SHARED((n_acc, d_row), jnp.float32),
            pltpu.VMEM((gb, _CHUNK), jnp.int32),
            pltpu.VMEM((gb, _CHUNK), jnp.int32),
            pltpu.VMEM((_CHUNK, d_row), jnp.float32),
            pltpu.VMEM((_CHUNK, d_row), jnp.float32),
            pltpu.SemaphoreType.DMA,
            pltpu.SemaphoreType.DMA,
            pltpu.SemaphoreType.DMA,
        ],
    )
    def k(h0, h1, src2d, dst2d, out0, out1, acc, idx_s, idx_d,
          rows0, rows1, sem_g0, sem_g1, sem_s):
        rows_bufs = (rows0, rows1)
        sem_g = (sem_g0, sem_g1)
        c = lax.axis_index("c")
        s = lax.axis_index("s")
        base = jnp.where(c == 0, s * cpt0, _SUBS * cpt0 + s * cpt1)
        n_groups = jnp.where(c == 0, cpt0 // gb, cpt1 // gb)

        def zb(r, carry):
            for u in range(d_row // 16):
                rows0[r, pl.ds(u * 16, 16)] = jnp.zeros((16,), jnp.float32)
            return carry
        lax.fori_loop(0, _CHUNK, zb, 0)
        for t in range(rows_per_sub // _CHUNK):
            pltpu.sync_copy(rows0,
                            acc.at[pl.ds(s * rows_per_sub + t * _CHUNK,
                                         _CHUNK)])

        plsc.subcore_barrier()

        def body(grp, carry):
            gbase = base + grp * gb
            pltpu.sync_copy(src2d.at[pl.ds(gbase, gb)], idx_s)
            pltpu.sync_copy(dst2d.at[pl.ds(gbase, gb)], idx_d)
            for j in range(2):
                pltpu.async_copy(h0.at[idx_s.at[j]], rows_bufs[j],
                                 sem_g[j])
            for j in range(gb):
                b = j % 2
                pltpu.make_async_copy(h0.at[idx_s.at[j]], rows_bufs[b],
                                      sem_g[b]).wait()
                pltpu.sync_copy(rows_bufs[b], acc.at[idx_d.at[j]],
                                add=True)
                if j + 2 < gb:
                    pltpu.async_copy(h0.at[idx_s.at[j + 2]],
                                     rows_bufs[b], sem_g[b])
            return carry

        lax.fori_loop(0, n_groups, body, 0)

        plsc.subcore_barrier()

        @pl.when(c == 0)
        def _():
            pltpu.sync_copy(acc.at[pl.ds(s * rows_per_sub, rows_per_sub)],
                            out0.at[pl.ds(s * rows_per_sub, rows_per_sub)])

        @pl.when(c == 1)
        def _():
            pltpu.sync_copy(acc.at[pl.ds(s * rows_per_sub, rows_per_sub)],
                            out1.at[pl.ds(s * rows_per_sub, rows_per_sub)])

    return k


def _tc_mlp(n, din, hmid, hout, extra_elu, block,
            first_matmul=True, post_w_cols=None):
    def body(*refs):
        if first_matmul:
            x, a, b, w1, b1, w2, b2 = refs[:7]
            nin = 7
        else:
            x, a, b, b1, w2, b2 = refs[:6]
            nin = 6
        xx = x[...] + a[...] + b[...]
        if first_matmul:
            t = jnp.dot(xx, w1[...], preferred_element_type=jnp.float32)
        else:
            t = xx
        h = _elu(t + b1[...])
        h = _elu(jnp.dot(h, w2[...], preferred_element_type=jnp.float32)
                 + b2[...])
        if extra_elu:
            h = _elu(h)
        if post_w_cols is not None:
            wp = refs[nin]
            h = jnp.dot(h, wp[...], preferred_element_type=jnp.float32)
        refs[-1][...] = h

    def row_spec(dcols):
        return pl.BlockSpec((block, dcols), lambda i: (i, 0))

    def full(shape):
        return pl.BlockSpec(shape, lambda i: (0,) * len(shape))

    in_specs = [row_spec(din)] * 3
    if first_matmul:
        in_specs += [full((din, hmid)), full((1, hmid))]
    else:
        in_specs += [full((1, hmid))]
    in_specs += [full((hmid, hout)), full((1, hout))]
    ocols = hout
    if post_w_cols is not None:
        in_specs.append(full((hout, post_w_cols)))
        ocols = post_w_cols

    return pl.pallas_call(
        body,
        grid=(n // block,),
        in_specs=in_specs,
        out_specs=row_spec(ocols),
        out_shape=jax.ShapeDtypeStruct((n, ocols), jnp.float32),
    )


def _tc_pool_mlp(n, d, g, dmid, block):
    nb = n // block
    dh = d // 2

    def body(hr, seg_ref, wm1, bm1, wm2, bm2, wm3, bm3, o, sums, counts):
        i = pl.program_id(0)

        @pl.when(i == 0)
        def _():
            sums[...] = jnp.zeros_like(sums)
            counts[...] = jnp.zeros_like(counts)

        seg = seg_ref[0, 0, :]
        oh = (seg[:, None]
              == lax.broadcasted_iota(jnp.int32, (block, g), 1)).astype(jnp.float32)
        h = hr[...]
        dn = (((0,), (0,)), ((), ()))
        sums[...] += lax.dot_general(oh, h, dn,
                                     preferred_element_type=jnp.float32)
        counts[...] += lax.dot_general(oh, jnp.ones((block, d), jnp.float32),
                                       dn, preferred_element_type=jnp.float32)

        @pl.when(i == nb - 1)
        def _():
            pooled = sums[...] / jnp.maximum(counts[...], 1.0)
            t = _elu(jnp.dot(pooled, wm1[...],
                                   preferred_element_type=jnp.float32) + bm1[...])
            t = _elu(jnp.dot(t, wm2[...],
                                   preferred_element_type=jnp.float32) + bm2[...])
            t = _elu(jnp.dot(t, wm3[...],
                                   preferred_element_type=jnp.float32) + bm3[...])
            o[...] = t

    def row_spec(dcols):
        return pl.BlockSpec((block, dcols), lambda i: (i, 0))

    def full(shape):
        return pl.BlockSpec(shape, lambda i: (0,) * len(shape))

    return pl.pallas_call(
        body,
        grid=(nb,),
        in_specs=[row_spec(d),
                  pl.BlockSpec((1, 1, block), lambda i: (i, 0, 0)),
                  full((d, dmid)), full((1, dmid)), full((dmid, d)),
                  full((1, d)), full((d, d)), full((1, d))],
        out_specs=pl.BlockSpec((g, d), lambda i: (0, 0)),
        out_shape=jax.ShapeDtypeStruct((g, d), jnp.float32),
        scratch_shapes=[pltpu.VMEM((g, d), jnp.float32),
                        pltpu.VMEM((g, d), jnp.float32)],
    )


def kernel(x, edge_index, batch,
           w1_1, b1_1, w1_2, b1_2,
           w2_1, b2_1, w2_2, b2_2,
           w3_1, b3_1, w3_2, b3_2,
           wm1, bm1, wm2, bm2, wm3, bm3):
    n, d = x.shape
    e = edge_index.shape[1]
    g = 64
    block = 1000

    ec = 2 * _SUBS * _CHUNK * 8
    epad = ((e + ec - 1) // ec) * ec
    cpt_sum = epad // (_SUBS * _CHUNK)
    cpt0 = cpt_sum - 8 * ((cpt_sum + 15) // 16)
    cpt1 = cpt_sum - cpt0
    nc = _SUBS * _CHUNK
    n_acc = ((n + nc - 1) // nc) * nc
    if n_acc == n:
        n_acc += nc
    pad_ar = jnp.arange(epad - e, dtype=jnp.int32)
    src_p = jnp.concatenate(
        [edge_index[0], pad_ar % n]).reshape(-1, _CHUNK)
    dst_p = jnp.concatenate(
        [edge_index[1], n + pad_ar % (n_acc - n)]).reshape(-1, _CHUNK)

    sc_full = _sc_scatter_add(n_acc, d, cpt0, cpt1)

    a1, p1 = sc_full(x, x, src_p, dst_p)
    h1 = _tc_mlp(n, 128, 128, 128, True, block)(
        x, a1, p1, w1_1, b1_1.reshape(1, -1), w1_2, b1_2.reshape(1, -1))

    a2, p2 = sc_full(h1, h1, src_p, dst_p)
    y2 = _tc_mlp(n, 128, 256, 256, True, block, post_w_cols=128)(
        h1, a2, p2, w2_1, b2_1.reshape(1, -1), w2_2, b2_2.reshape(1, -1),
        w3_1)

    a3, p3 = sc_full(y2, y2, src_p, dst_p)
    h3 = _tc_mlp(n, 128, 128, 128, False, block, first_matmul=False)(
        y2, a3, p3, b3_1.reshape(1, -1), w3_2, b3_2.reshape(1, -1))

    o = _tc_pool_mlp(n, 128, g, 256, block)(
        h3, batch.reshape(n // block, 1, block),
        wm1, bm1.reshape(1, -1), wm2, bm2.reshape(1, -1),
        wm3, bm3.reshape(1, -1))
    return o
